# baseline (speedup 1.0000x reference)
import functools
import math

import jax
import jax.numpy as jnp
from jax.experimental import pallas as pl
from jax.experimental.pallas import tpu as pltpu

B = 4096
D = 512
NUM_CLASSES = 100000
M = 0.1
S = 1.0
EPSILON = 0.1

ROWS_PER_GATHER = 8
ROW_TILE = 512


def _gather_norm_kernel(lbl_ref, *refs):
    out_ref = refs[-1]
    rows = jnp.concatenate([r[...].reshape(1, D) for r in refs[:-1]], axis=0)
    norm = jnp.sqrt(jnp.sum(rows * rows, axis=1, keepdims=True))
    out_ref[...] = rows / norm


def _loss_kernel(lbl_col_ref, lbl_row_ref, x_ref, cn_ref, out_ref,
                 inv_c_ref):
    i = pl.program_id(0)

    @pl.when(i == 0)
    def _():
        eq = (lbl_col_ref[...] == lbl_row_ref[...]).astype(jnp.float32)
        inv_c_ref[...] = 1.0 / jnp.sum(eq, axis=0, keepdims=True)

    inv_c = inv_c_ref[...]
    k = jnp.sum(inv_c)

    xt = x_ref[...]
    norm = jnp.sqrt(jnp.sum(xt * xt, axis=1, keepdims=True))
    xn = xt / jnp.maximum(norm, 1e-12)

    a = jax.lax.dot_general(xn, cn_ref[...], (((1,), (1,)), ((), ())),
                            preferred_element_type=jnp.float32)
    a = jnp.clip(a, -1.0, 1.0)
    w = S * a
    ew = jnp.exp(w)
    sumexp = jnp.sum(ew * inv_c, axis=1, keepdims=True)
    sumw = jnp.sum(w * inv_c, axis=1, keepdims=True)

    ri = jax.lax.broadcasted_iota(jnp.int32, (ROW_TILE, B), 0)
    ci = jax.lax.broadcasted_iota(jnp.int32, (ROW_TILE, B), 1)
    t = jnp.sum(jnp.where(ci == ri + i * ROW_TILE, a, 0.0),
                axis=1, keepdims=True)

    tm = t * math.cos(M) - jnp.sqrt(jnp.maximum(1.0 - t * t, 0.0)) * math.sin(M)
    z = sumexp - jnp.exp(S * t) + jnp.exp(S * tm)
    sw = sumw - S * t + S * tm
    logz = jnp.log(z)

    row_loss = (-(1.0 - EPSILON) * (S * tm - logz)
                - (EPSILON / NUM_CLASSES) * (sw - k * logz))
    partial = jnp.sum(row_loss, keepdims=True).reshape(1, 1) * (1.0 / B)

    @pl.when(i == 0)
    def _():
        out_ref[...] = partial

    @pl.when(i != 0)
    def _():
        out_ref[...] += partial


def kernel(x, labels, W):
    n_steps = B // ROWS_PER_GATHER
    W3 = W.reshape(NUM_CLASSES, 1, D)
    in_specs = [
        pl.BlockSpec((1, 1, D),
                     functools.partial(
                         lambda r, i, lbl: (lbl[i * ROWS_PER_GATHER + r], 0, 0),
                         r))
        for r in range(ROWS_PER_GATHER)
    ]
    cn = pl.pallas_call(
        _gather_norm_kernel,
        grid_spec=pltpu.PrefetchScalarGridSpec(
            num_scalar_prefetch=1,
            grid=(n_steps,),
            in_specs=[in_specs[r] for r in range(ROWS_PER_GATHER)],
            out_specs=pl.BlockSpec((ROWS_PER_GATHER, D), lambda i, lbl: (i, 0)),
        ),
        out_shape=jax.ShapeDtypeStruct((B, D), jnp.float32),
    )(labels, *([W3] * ROWS_PER_GATHER))

    lbl_col = labels.reshape(B, 1)
    lbl_row = labels.reshape(1, B)
    out = pl.pallas_call(
        _loss_kernel,
        grid=(B // ROW_TILE,),
        in_specs=[
            pl.BlockSpec((B, 1), lambda i: (0, 0)),
            pl.BlockSpec((1, B), lambda i: (0, 0)),
            pl.BlockSpec((ROW_TILE, D), lambda i: (i, 0)),
            pl.BlockSpec((B, D), lambda i: (0, 0)),
        ],
        out_specs=pl.BlockSpec((1, 1), lambda i: (0, 0)),
        out_shape=jax.ShapeDtypeStruct((1, 1), jnp.float32),
        scratch_shapes=[pltpu.VMEM((1, B), jnp.float32)],
    )(lbl_col, lbl_row, x, cn)
    return out[0, 0]


# speedup vs baseline: 1.0606x; 1.0606x over previous
import functools
import math

import jax
import jax.numpy as jnp
from jax.experimental import pallas as pl
from jax.experimental.pallas import tpu as pltpu

B = 4096
D = 512
NUM_CLASSES = 100000
M = 0.1
S = 1.0
EPSILON = 0.1

ROWS_PER_GATHER = 8
ROW_TILE = 512
CNT_TILE = 512


def _gather_kernel(lbl_ref, *refs):
    out_ref = refs[-1]
    rows = jnp.concatenate([r[...].reshape(1, D) for r in refs[:-1]], axis=0)
    out_ref[...] = rows


def _counts_kernel(lbl_col_ref, lbl_row_ref, out_ref):
    eq = jnp.where(lbl_col_ref[...] == lbl_row_ref[...], 1.0, 0.0)
    out_ref[...] = 1.0 / jnp.sum(eq, axis=0, keepdims=True)


def _loss_kernel(x_ref, cn_ref, inv_c_ref, out_ref, cnn_ref, cnbf_ref):
    i = pl.program_id(0)

    @pl.when(i == 0)
    def _():
        c = cn_ref[...]
        norm = jnp.sqrt(jnp.sum(c * c, axis=1, keepdims=True))
        cnn = c / norm
        cnn_ref[...] = cnn
        cnbf_ref[...] = cnn.astype(jnp.bfloat16)

    inv_c = inv_c_ref[...]
    k = jnp.sum(inv_c)

    xt = x_ref[...]
    norm = jnp.sqrt(jnp.sum(xt * xt, axis=1, keepdims=True))
    xn = xt / jnp.maximum(norm, 1e-12)

    cnt = cnn_ref[pl.ds(i * ROW_TILE, ROW_TILE), :]
    t = jnp.sum(xn * cnt, axis=1, keepdims=True)
    t = jnp.clip(t, -1.0, 1.0)
    tm = t * math.cos(M) - jnp.sqrt(jnp.maximum(1.0 - t * t, 0.0)) * math.sin(M)

    a = jax.lax.dot_general(xn.astype(jnp.bfloat16), cnbf_ref[...],
                            (((1,), (1,)), ((), ())),
                            preferred_element_type=jnp.float32)
    a = jnp.clip(a, -1.0, 1.0)
    ew = jnp.exp(S * a)
    sumexp = jnp.sum(ew * inv_c, axis=1, keepdims=True)
    sumw = S * jnp.sum(a * inv_c, axis=1, keepdims=True)

    z = sumexp - jnp.exp(S * t) + jnp.exp(S * tm)
    sw = sumw - S * t + S * tm
    logz = jnp.log(z)

    row_loss = (-(1.0 - EPSILON) * (S * tm - logz)
                - (EPSILON / NUM_CLASSES) * (sw - k * logz))
    partial = jnp.sum(row_loss, keepdims=True).reshape(1, 1) * (1.0 / B)

    @pl.when(i == 0)
    def _():
        out_ref[...] = partial

    @pl.when(i != 0)
    def _():
        out_ref[...] += partial


def kernel(x, labels, W):
    n_steps = B // ROWS_PER_GATHER
    W3 = W.reshape(NUM_CLASSES, 1, D)
    in_specs = [
        pl.BlockSpec((1, 1, D),
                     functools.partial(
                         lambda r, i, lbl: (lbl[i * ROWS_PER_GATHER + r], 0, 0),
                         r))
        for r in range(ROWS_PER_GATHER)
    ]
    cn = pl.pallas_call(
        _gather_kernel,
        grid_spec=pltpu.PrefetchScalarGridSpec(
            num_scalar_prefetch=1,
            grid=(n_steps,),
            in_specs=[in_specs[r] for r in range(ROWS_PER_GATHER)],
            out_specs=pl.BlockSpec((ROWS_PER_GATHER, D), lambda i, lbl: (i, 0)),
        ),
        out_shape=jax.ShapeDtypeStruct((B, D), jnp.float32),
    )(labels, *([W3] * ROWS_PER_GATHER))

    lbl_col = labels.reshape(B, 1)
    lbl_row = labels.reshape(1, B)
    inv_c = pl.pallas_call(
        _counts_kernel,
        grid=(B // CNT_TILE,),
        in_specs=[
            pl.BlockSpec((B, 1), lambda j: (0, 0)),
            pl.BlockSpec((1, CNT_TILE), lambda j: (0, j)),
        ],
        out_specs=pl.BlockSpec((1, CNT_TILE), lambda j: (0, j)),
        out_shape=jax.ShapeDtypeStruct((1, B), jnp.float32),
    )(lbl_col, lbl_row)

    out = pl.pallas_call(
        _loss_kernel,
        grid=(B // ROW_TILE,),
        in_specs=[
            pl.BlockSpec((ROW_TILE, D), lambda i: (i, 0)),
            pl.BlockSpec((B, D), lambda i: (0, 0)),
            pl.BlockSpec((1, B), lambda i: (0, 0)),
        ],
        out_specs=pl.BlockSpec((1, 1), lambda i: (0, 0)),
        out_shape=jax.ShapeDtypeStruct((1, 1), jnp.float32),
        scratch_shapes=[pltpu.VMEM((B, D), jnp.float32),
                        pltpu.VMEM((B, D), jnp.bfloat16)],
    )(x, cn, inv_c)
    return out[0, 0]


# speedup vs baseline: 7.7393x; 7.2972x over previous
import functools
import math

import jax
import jax.numpy as jnp
from jax import lax
from jax.experimental import pallas as pl
from jax.experimental.pallas import tpu as pltpu
from jax.experimental.pallas import tpu_sc as plsc

B = 4096
D = 512
NUM_CLASSES = 100000
M = 0.1
S = 1.0
EPSILON = 0.1

ROWS_PER_GATHER = 8
ROW_TILE = 512
CNT_TILE = 512


def _counts_kernel(lbl_col_ref, lbl_row_ref, out_ref):
    eq = jnp.where(lbl_col_ref[...] == lbl_row_ref[...], 1.0, 0.0)
    out_ref[...] = 1.0 / jnp.sum(eq, axis=0, keepdims=True)


def _loss_kernel(x_ref, cn_ref, inv_c_ref, out_ref, cnn_ref, cnbf_ref):
    i = pl.program_id(0)

    @pl.when(i == 0)
    def _():
        c = cn_ref[...]
        norm = jnp.sqrt(jnp.sum(c * c, axis=1, keepdims=True))
        cnn = c / norm
        cnn_ref[...] = cnn
        cnbf_ref[...] = cnn.astype(jnp.bfloat16)

    inv_c = inv_c_ref[...]
    k = jnp.sum(inv_c)

    xt = x_ref[...]
    norm = jnp.sqrt(jnp.sum(xt * xt, axis=1, keepdims=True))
    xn = xt / jnp.maximum(norm, 1e-12)

    cnt = cnn_ref[pl.ds(i * ROW_TILE, ROW_TILE), :]
    t = jnp.sum(xn * cnt, axis=1, keepdims=True)
    t = jnp.clip(t, -1.0, 1.0)
    tm = t * math.cos(M) - jnp.sqrt(jnp.maximum(1.0 - t * t, 0.0)) * math.sin(M)

    a = jax.lax.dot_general(xn.astype(jnp.bfloat16), cnbf_ref[...],
                            (((1,), (1,)), ((), ())),
                            preferred_element_type=jnp.float32)
    a = jnp.clip(a, -1.0, 1.0)
    ew = jnp.exp(S * a)
    sumexp = jnp.sum(ew * inv_c, axis=1, keepdims=True)
    sumw = S * jnp.sum(a * inv_c, axis=1, keepdims=True)

    z = sumexp - jnp.exp(S * t) + jnp.exp(S * tm)
    sw = sumw - S * t + S * tm
    logz = jnp.log(z)

    row_loss = (-(1.0 - EPSILON) * (S * tm - logz)
                - (EPSILON / NUM_CLASSES) * (sw - k * logz))
    partial = jnp.sum(row_loss, keepdims=True).reshape(1, 1) * (1.0 / B)

    @pl.when(i == 0)
    def _():
        out_ref[...] = partial

    @pl.when(i != 0)
    def _():
        out_ref[...] += partial


def _sc_gather(labels, W):
    info = plsc.get_sparse_core_info()
    nw = info.num_cores * info.num_subcores
    b_per_w = B // nw
    mesh = plsc.VectorSubcoreMesh(core_axis_name="c", subcore_axis_name="s")

    @functools.partial(
        pl.kernel, mesh=mesh,
        out_type=jax.ShapeDtypeStruct((B, D), jnp.float32),
        scratch_types=[
            pltpu.VMEM((b_per_w,), jnp.int32),
            pltpu.VMEM((b_per_w, D), jnp.float32),
            pltpu.SemaphoreType.DMA,
        ],
    )
    def gather(table_hbm, idx_hbm, out_hbm, idx_v, rows_v, sem):
        wid = lax.axis_index("s") * info.num_cores + lax.axis_index("c")
        base = wid * b_per_w
        pltpu.sync_copy(idx_hbm.at[pl.ds(base, b_per_w)], idx_v)
        pltpu.async_copy(table_hbm.at[idx_v], rows_v, sem).wait()
        pltpu.sync_copy(rows_v, out_hbm.at[pl.ds(base, b_per_w)])

    return gather(W, labels)


def kernel(x, labels, W):
    cn = _sc_gather(labels, W)

    lbl_col = labels.reshape(B, 1)
    lbl_row = labels.reshape(1, B)
    inv_c = pl.pallas_call(
        _counts_kernel,
        grid=(B // CNT_TILE,),
        in_specs=[
            pl.BlockSpec((B, 1), lambda j: (0, 0)),
            pl.BlockSpec((1, CNT_TILE), lambda j: (0, j)),
        ],
        out_specs=pl.BlockSpec((1, CNT_TILE), lambda j: (0, j)),
        out_shape=jax.ShapeDtypeStruct((1, B), jnp.float32),
    )(lbl_col, lbl_row)

    out = pl.pallas_call(
        _loss_kernel,
        grid=(B // ROW_TILE,),
        in_specs=[
            pl.BlockSpec((ROW_TILE, D), lambda i: (i, 0)),
            pl.BlockSpec((B, D), lambda i: (0, 0)),
            pl.BlockSpec((1, B), lambda i: (0, 0)),
        ],
        out_specs=pl.BlockSpec((1, 1), lambda i: (0, 0)),
        out_shape=jax.ShapeDtypeStruct((1, 1), jnp.float32),
        scratch_shapes=[pltpu.VMEM((B, D), jnp.float32),
                        pltpu.VMEM((B, D), jnp.bfloat16)],
    )(x, cn, inv_c)
    return out[0, 0]
